# Initial kernel scaffold; baseline (speedup 1.0000x reference)
import jax
import jax.numpy as jnp
from jax.experimental import pallas as pl


def kernel(seq_word, seq_pos, seq_ner, word_table, pos_table, ner_table):
    raise NotImplementedError("write your pallas kernel here")


# speedup vs baseline: 1.4061x; 1.4061x over previous
import functools

import jax
import jax.numpy as jnp
from jax import lax
from jax.experimental import pallas as pl
from jax.experimental.pallas import tpu as pltpu
from jax.experimental.pallas import tpu_sc as plsc

S_LEN = 200
BATCH = 1024
N_TOK = S_LEN * BATCH
D_WORD = 64
D_TAG = 16
D_OUT = D_WORD + 2 * D_TAG

NUM_CORES = 2
NUM_SUBCORES = 16
NW = NUM_CORES * NUM_SUBCORES
TOK_PER_W = N_TOK // NW
SUB = 128
CHUNK = 640
KSUB = CHUNK // SUB
NCHUNK = TOK_PER_W // CHUNK
ROWS_PER_W = TOK_PER_W // SUB


def _sc_embed(word_table, pos_table, ner_table, idxw, idxp, idxn):
    mesh = plsc.VectorSubcoreMesh(core_axis_name="c", subcore_axis_name="s")

    @functools.partial(
        pl.kernel,
        out_type=jax.ShapeDtypeStruct((N_TOK, D_OUT), jnp.float32),
        mesh=mesh,
        scratch_types=[
            pltpu.VMEM((KSUB, SUB), jnp.int32),
            pltpu.VMEM((KSUB, SUB), jnp.int32),
            pltpu.VMEM((KSUB, SUB), jnp.int32),
            pltpu.VMEM((CHUNK, D_WORD), jnp.float32),
            pltpu.VMEM((CHUNK, D_TAG), jnp.float32),
            pltpu.VMEM((CHUNK, D_TAG), jnp.float32),
            pltpu.SemaphoreType.DMA,
        ],
        compiler_params=pltpu.CompilerParams(use_tc_tiling_on_sc=False),
    )
    def k(wt, pt, nt, iw, ip, inr, out, iw_v, ip_v, in_v, wrow, prow, nrow, sem):
        wid = lax.axis_index("s") * NUM_CORES + lax.axis_index("c")
        base_row = wid * ROWS_PER_W
        base_tok = wid * TOK_PER_W

        def body(ci, carry):
            row0 = base_row + ci * KSUB
            t0 = base_tok + ci * CHUNK
            pltpu.sync_copy(iw.at[pl.ds(row0, KSUB)], iw_v)
            pltpu.sync_copy(ip.at[pl.ds(row0, KSUB)], ip_v)
            pltpu.sync_copy(inr.at[pl.ds(row0, KSUB)], in_v)
            copies = []
            for j in range(KSUB):
                dst = wrow.at[pl.ds(j * SUB, SUB)]
                c = pltpu.make_async_copy(wt.at[iw_v.at[j]], dst, sem)
                c.start()
                copies.append(c)
                dst = prow.at[pl.ds(j * SUB, SUB)]
                c = pltpu.make_async_copy(pt.at[ip_v.at[j]], dst, sem)
                c.start()
                copies.append(c)
                dst = nrow.at[pl.ds(j * SUB, SUB)]
                c = pltpu.make_async_copy(nt.at[in_v.at[j]], dst, sem)
                c.start()
                copies.append(c)
            for c in copies:
                c.wait()
            pltpu.sync_copy(wrow, out.at[pl.ds(t0, CHUNK), pl.ds(0, D_WORD)])
            pltpu.sync_copy(prow, out.at[pl.ds(t0, CHUNK), pl.ds(D_WORD, D_TAG)])
            pltpu.sync_copy(nrow, out.at[pl.ds(t0, CHUNK), pl.ds(D_WORD + D_TAG, D_TAG)])
            return carry

        lax.fori_loop(0, NCHUNK, body, 0)

    return k(word_table, pos_table, ner_table, idxw, idxp, idxn)


def kernel(seq_word, seq_pos, seq_ner, word_table, pos_table, ner_table):
    idxw = seq_word.reshape(N_TOK // SUB, SUB).astype(jnp.int32)
    idxp = seq_pos.reshape(N_TOK // SUB, SUB).astype(jnp.int32)
    idxn = seq_ner.reshape(N_TOK // SUB, SUB).astype(jnp.int32)
    out = _sc_embed(word_table, pos_table, ner_table, idxw, idxp, idxn)
    return out.reshape(S_LEN, BATCH, D_OUT)


# speedup vs baseline: 2.0180x; 1.4352x over previous
import functools

import jax
import jax.numpy as jnp
from jax import lax
from jax.experimental import pallas as pl
from jax.experimental.pallas import tpu as pltpu
from jax.experimental.pallas import tpu_sc as plsc

S_LEN = 200
BATCH = 1024
N_TOK = S_LEN * BATCH
D_WORD = 64
D_TAG = 16
D_CROSS = 2 * D_TAG
D_OUT = D_WORD + D_CROSS
POS_DICT = 50
NER_DICT = 20

NUM_CORES = 2
NUM_SUBCORES = 16
NW = NUM_CORES * NUM_SUBCORES
TOK_PER_W = N_TOK // NW
SUB = 128
NCH = TOK_PER_W // SUB
RING = 6
DEPTH = 4
LANES = 16


def _sc_embed(word_table, cross_table, idxw, idxp, idxn):
    mesh = plsc.VectorSubcoreMesh(core_axis_name="c", subcore_axis_name="s")

    @functools.partial(
        pl.kernel,
        out_type=jax.ShapeDtypeStruct((N_TOK, D_OUT), jnp.float32),
        mesh=mesh,
        scratch_types=[
            pltpu.VMEM((NCH, SUB), jnp.int32),
            pltpu.VMEM((NCH, SUB), jnp.int32),
            pltpu.VMEM((NCH, SUB), jnp.int32),
            pltpu.VMEM((NCH, SUB), jnp.int32),
            pltpu.VMEM((RING, SUB, D_WORD), jnp.float32),
            pltpu.VMEM((RING, SUB, D_CROSS), jnp.float32),
            pltpu.SemaphoreType.DMA,
            pltpu.SemaphoreType.DMA,
        ],
        compiler_params=pltpu.CompilerParams(use_tc_tiling_on_sc=False),
    )
    def k(wt, ct, iw, ip, inr, out, iw_v, ip_v, in_v, it_v, wbuf, tbuf, gsem, wsem):
        wid = lax.axis_index("s") * NUM_CORES + lax.axis_index("c")
        base_tok = wid * TOK_PER_W

        pltpu.sync_copy(iw.at[wid], iw_v)
        pltpu.sync_copy(ip.at[wid], ip_v)
        pltpu.sync_copy(inr.at[wid], in_v)

        def tag_body(r, c):
            for g in range(SUB // LANES):
                p = ip_v[r, pl.ds(g * LANES, LANES)]
                n = in_v[r, pl.ds(g * LANES, LANES)]
                it_v[r, pl.ds(g * LANES, LANES)] = p * NER_DICT + n
            return c

        lax.fori_loop(0, NCH, tag_body, 0)

        def fire(cg, slot):
            pltpu.make_async_copy(wt.at[iw_v.at[cg]], wbuf.at[slot], gsem).start()
            pltpu.make_async_copy(ct.at[it_v.at[cg]], tbuf.at[slot], gsem).start()

        def write_descs(slot, t0):
            return (
                pltpu.make_async_copy(
                    wbuf.at[slot], out.at[pl.ds(t0, SUB), pl.ds(0, D_WORD)], wsem
                ),
                pltpu.make_async_copy(
                    tbuf.at[slot], out.at[pl.ds(t0, SUB), pl.ds(D_WORD, D_CROSS)], wsem
                ),
            )

        for cg in range(DEPTH):
            fire(cg, cg)

        def body(ci, c):
            cg = ci + DEPTH
            slot_g = lax.rem(cg, RING)

            @pl.when(jnp.logical_and(cg < NCH, cg >= RING))
            def _():
                for d in write_descs(slot_g, base_tok):
                    d.wait()

            @pl.when(cg < NCH)
            def _():
                fire(cg, slot_g)

            slot = lax.rem(ci, RING)
            t0 = base_tok + ci * SUB
            pltpu.make_async_copy(wt.at[iw_v.at[ci]], wbuf.at[slot], gsem).wait()
            pltpu.make_async_copy(ct.at[it_v.at[ci]], tbuf.at[slot], gsem).wait()
            for d in write_descs(slot, t0):
                d.start()
            return c

        lax.fori_loop(0, NCH, body, 0)

        for _i in range(RING):
            for d in write_descs(0, base_tok):
                d.wait()

    return k(word_table, cross_table, idxw, idxp, idxn)


def kernel(seq_word, seq_pos, seq_ner, word_table, pos_table, ner_table):
    cross = jnp.concatenate(
        [
            jnp.broadcast_to(pos_table[:, None, :], (POS_DICT, NER_DICT, D_TAG)),
            jnp.broadcast_to(ner_table[None, :, :], (POS_DICT, NER_DICT, D_TAG)),
        ],
        axis=2,
    ).reshape(POS_DICT * NER_DICT, D_CROSS)
    idxw = seq_word.reshape(NW, NCH, SUB).astype(jnp.int32)
    idxp = seq_pos.reshape(NW, NCH, SUB).astype(jnp.int32)
    idxn = seq_ner.reshape(NW, NCH, SUB).astype(jnp.int32)
    out = _sc_embed(word_table, cross, idxw, idxp, idxn)
    return out.reshape(S_LEN, BATCH, D_OUT)
